# baseline (speedup 1.0000x reference)
import functools

import jax
import jax.numpy as jnp
from jax import lax
from jax.experimental import pallas as pl
from jax.experimental.pallas import tpu as pltpu
from jax.experimental.pallas import tpu_sc as plsc

VOCAB = 1000000
DIM = 64
BATCH = 16384
HIST = 50
B = BATCH * HIST

NUM_CORES = 2
NUM_SUBCORES = 16
NW = NUM_CORES * NUM_SUBCORES
BATCH_PW = BATCH // NW

CB = 16
IDXW = 64
HP = 56
DP = 128
NCHUNK = BATCH_PW // CB

_mesh = plsc.VectorSubcoreMesh(core_axis_name="c", subcore_axis_name="s")


@functools.partial(
    pl.kernel,
    mesh=_mesh,
    compiler_params=pltpu.CompilerParams(use_tc_tiling_on_sc=False),
    out_type=jax.ShapeDtypeStruct((BATCH, HP, DP), jnp.float32),
    scratch_types=[
        pltpu.VMEM((CB * IDXW,), jnp.int32),
        pltpu.VMEM((CB * IDXW,), jnp.int32),
        pltpu.VMEM((CB, HP, DIM), jnp.float32),
        pltpu.VMEM((CB, HP, DIM), jnp.float32),
        pltpu.SemaphoreType.DMA,
        pltpu.SemaphoreType.DMA,
        pltpu.SemaphoreType.DMA,
        pltpu.SemaphoreType.DMA,
        pltpu.SemaphoreType.DMA,
        pltpu.SemaphoreType.DMA,
    ],
)
def _gather(idx_hbm, table_hbm, out_hbm, idxc0, idxc1, rows0, rows1,
            i0, i1, g0, g1, s0, s1):
    wid = lax.axis_index("s") * NUM_CORES + lax.axis_index("c")
    wb0 = wid * BATCH_PW

    idxc = (idxc0, idxc1)
    rows = (rows0, rows1)
    isem = (i0, i1)
    gsem = (g0, g1)
    ssem = (s0, s1)

    def idx_copy(c, b):
        return pltpu.make_async_copy(
            idx_hbm.at[pl.ds((wb0 + c * CB) * IDXW, CB * IDXW)],
            idxc[b],
            isem[b],
        )

    def issue_gathers(b):
        for jb in range(CB):
            pltpu.async_copy(
                table_hbm.at[idxc[b].at[pl.ds(jb * IDXW, HP)]],
                rows[b].at[jb],
                gsem[b],
            )

    def wait_gathers(b):
        for jb in range(CB):
            pltpu.make_async_copy(
                table_hbm.at[idxc[b].at[pl.ds(jb * IDXW, HP)]],
                rows[b].at[jb],
                gsem[b],
            ).wait()

    def issue_stores(c, b):
        for jb in range(CB):
            pltpu.async_copy(
                rows[b].at[jb],
                out_hbm.at[wb0 + c * CB + jb].at[:, pl.ds(0, DIM)],
                ssem[b],
            )

    def wait_store(b):
        for jb in range(CB):
            pltpu.make_async_copy(
                rows[b].at[jb],
                out_hbm.at[wb0 + jb].at[:, pl.ds(0, DIM)],
                ssem[b],
            ).wait()

    c0 = idx_copy(0, 0)
    c0.start()
    c0.wait()
    idx_copy(1, 1).start()
    issue_gathers(0)

    def outer(i, carry):
        for b in range(2):
            c = 2 * i + b
            nb = 1 - b
            wait_gathers(b)
            issue_stores(c, b)

            @pl.when(c > 0)
            def _():
                wait_store(nb)

            @pl.when(c + 1 < NCHUNK)
            def _():
                idx_copy(0, nb).wait()
                issue_gathers(nb)

            @pl.when(c + 2 < NCHUNK)
            def _():
                idx_copy(c + 2, b).start()

        return carry

    lax.fori_loop(0, NCHUNK // 2, outer, 0)
    wait_store(1)


def kernel(indices, table):
    idxp = jnp.pad(indices, ((0, 0), (0, IDXW - HIST))).reshape(BATCH * IDXW)
    out = _gather(idxp, table)
    return out[:, :HIST, :DIM]


# speedup vs baseline: 1.0013x; 1.0013x over previous
import functools

import jax
import jax.numpy as jnp
from jax import lax
from jax.experimental import pallas as pl
from jax.experimental.pallas import tpu as pltpu
from jax.experimental.pallas import tpu_sc as plsc

VOCAB = 1000000
DIM = 64
BATCH = 16384
HIST = 50
B = BATCH * HIST

NUM_CORES = 2
NUM_SUBCORES = 16
NW = NUM_CORES * NUM_SUBCORES
BATCH_PW = BATCH // NW

CB = 16
IDXW = 64
HP = 56
DP = 128
NCHUNK = BATCH_PW // CB

_mesh = plsc.VectorSubcoreMesh(core_axis_name="c", subcore_axis_name="s")


@functools.partial(
    pl.kernel,
    mesh=_mesh,
    compiler_params=pltpu.CompilerParams(use_tc_tiling_on_sc=False),
    out_type=jax.ShapeDtypeStruct((BATCH, HP, DP), jnp.float32),
    scratch_types=[
        pltpu.VMEM((CB * IDXW,), jnp.int32),
        pltpu.VMEM((CB * IDXW,), jnp.int32),
        pltpu.VMEM((CB, HP, DIM), jnp.float32),
        pltpu.VMEM((CB, HP, DIM), jnp.float32),
        pltpu.SemaphoreType.DMA,
        pltpu.SemaphoreType.DMA,
        pltpu.SemaphoreType.DMA,
        pltpu.SemaphoreType.DMA,
        pltpu.SemaphoreType.DMA,
        pltpu.SemaphoreType.DMA,
    ],
)
def _gather(idx_hbm, table_hbm, out_hbm, idxc0, idxc1, rows0, rows1,
            i0, i1, g0, g1, s0, s1):
    wid = lax.axis_index("s") * NUM_CORES + lax.axis_index("c")
    wb0 = wid * BATCH_PW

    idxc = (idxc0, idxc1)
    rows = (rows0, rows1)
    isem = (i0, i1)
    gsem = (g0, g1)
    ssem = (s0, s1)

    def idx_copy(c, b):
        return pltpu.make_async_copy(
            idx_hbm.at[pl.ds((wb0 + c * CB) * IDXW, CB * IDXW)],
            idxc[b],
            isem[b],
        )

    def issue_gathers(b):
        for jb in range(CB):
            pltpu.async_copy(
                table_hbm.at[idxc[b].at[pl.ds(jb * IDXW, HP)]],
                rows[b].at[jb],
                gsem[b],
            )

    def wait_gathers(b):
        for jb in range(CB):
            pltpu.make_async_copy(
                table_hbm.at[idxc[b].at[pl.ds(jb * IDXW, HP)]],
                rows[b].at[jb],
                gsem[b],
            ).wait()

    def issue_stores(c, b):
        pltpu.async_copy(
            rows[b],
            out_hbm.at[pl.ds(wb0 + c * CB, CB), pl.ds(0, HP), pl.ds(0, DIM)],
            ssem[b],
        )

    def wait_store(b):
        pltpu.make_async_copy(
            rows[b],
            out_hbm.at[pl.ds(wb0, CB), pl.ds(0, HP), pl.ds(0, DIM)],
            ssem[b],
        ).wait()

    c0 = idx_copy(0, 0)
    c0.start()
    c0.wait()
    idx_copy(1, 1).start()
    issue_gathers(0)

    def outer(i, carry):
        for b in range(2):
            c = 2 * i + b
            nb = 1 - b
            wait_gathers(b)
            issue_stores(c, b)

            @pl.when(c > 0)
            def _():
                wait_store(nb)

            @pl.when(c + 1 < NCHUNK)
            def _():
                idx_copy(0, nb).wait()
                issue_gathers(nb)

            @pl.when(c + 2 < NCHUNK)
            def _():
                idx_copy(c + 2, b).start()

        return carry

    lax.fori_loop(0, NCHUNK // 2, outer, 0)
    wait_store(1)


def kernel(indices, table):
    idxp = jnp.pad(indices, ((0, 0), (0, IDXW - HIST))).reshape(BATCH * IDXW)
    out = _gather(idxp, table)
    return out[:, :HIST, :DIM]


# speedup vs baseline: 2.4810x; 2.4779x over previous
import functools

import jax
import jax.numpy as jnp
from jax import lax
from jax.experimental import pallas as pl
from jax.experimental.pallas import tpu as pltpu
from jax.experimental.pallas import tpu_sc as plsc

VOCAB = 1000000
DIM = 64
BATCH = 16384
HIST = 50
B = BATCH * HIST

NUM_CORES = 2
NUM_SUBCORES = 16
NW = NUM_CORES * NUM_SUBCORES
BPW = B // NW

CHUNK = 512
SUB = 128
NSUB = CHUNK // SUB
NCHUNK = BPW // CHUNK

_mesh = plsc.VectorSubcoreMesh(core_axis_name="c", subcore_axis_name="s")


@functools.partial(
    pl.kernel,
    mesh=_mesh,
    compiler_params=pltpu.CompilerParams(use_tc_tiling_on_sc=False),
    out_type=jax.ShapeDtypeStruct((B, DIM), jnp.float32),
    scratch_types=[
        pltpu.VMEM((BPW,), jnp.int32),
        pltpu.VMEM((CHUNK, DIM), jnp.float32),
        pltpu.VMEM((CHUNK, DIM), jnp.float32),
        pltpu.SemaphoreType.DMA,
        pltpu.SemaphoreType.DMA,
        pltpu.SemaphoreType.DMA,
        pltpu.SemaphoreType.DMA,
    ],
)
def _gather(idx_hbm, table_hbm, out_hbm, idx_v, rows0, rows1, g0, g1, s0, s1):
    wid = lax.axis_index("s") * NUM_CORES + lax.axis_index("c")
    base = wid * BPW
    pltpu.sync_copy(idx_hbm.at[pl.ds(base, BPW)], idx_v)

    rows = (rows0, rows1)
    gsem = (g0, g1)
    ssem = (s0, s1)

    def issue_gathers(c, b):
        off = c * CHUNK
        for j in range(NSUB):
            pltpu.async_copy(
                table_hbm.at[idx_v.at[pl.ds(off + j * SUB, SUB)]],
                rows[b].at[pl.ds(j * SUB, SUB)],
                gsem[b],
            )

    def wait_gathers(b):
        for j in range(NSUB):
            pltpu.make_async_copy(
                table_hbm.at[idx_v.at[pl.ds(j * SUB, SUB)]],
                rows[b].at[pl.ds(j * SUB, SUB)],
                gsem[b],
            ).wait()

    def wait_store(b):
        pltpu.make_async_copy(
            rows[b], out_hbm.at[pl.ds(base, CHUNK)], ssem[b]
        ).wait()

    issue_gathers(0, 0)

    def outer(i, carry):
        for b in range(2):
            c = 2 * i + b
            nb = 1 - b
            wait_gathers(b)
            pltpu.async_copy(
                rows[b], out_hbm.at[pl.ds(base + c * CHUNK, CHUNK)], ssem[b]
            )

            @pl.when(c > 0)
            def _():
                wait_store(nb)

            @pl.when(c + 1 < NCHUNK)
            def _():
                issue_gathers(c + 1, nb)

        return carry

    lax.fori_loop(0, NCHUNK // 2, outer, 0)
    wait_store(1)


def kernel(indices, table):
    flat = indices.reshape(B)
    out = _gather(flat, table)
    return out.reshape(BATCH, HIST, DIM)
